# baseline (speedup 1.0000x reference)
import functools

import jax
import jax.numpy as jnp
from jax.experimental import pallas as pl
from jax.experimental.pallas import tpu as pltpu


def _bdot(a, b):
    return jnp.dot(a.astype(jnp.bfloat16), b.astype(jnp.bfloat16),
                   preferred_element_type=jnp.float32)


def _pick_block(n, cap, step=8):
    best = 8
    for d in range(step, cap + 1, step):
        if n % d == 0:
            best = d
    return best


def _mega_body(x_ref, adj_ref, w1_ref, b1_ref, w2_ref, b2_ref,
               w3_ref, lw1_ref, lw2_ref, lw3_ref, b3_ref, linb_ref,
               out_ref, ahbm_ref, s1t3_scr, s2_scr, p12_scr, avm_scr,
               wsem, rsem, *, ni, nj, bf, bs, nclass):
    s = pl.program_id(0)
    bf16 = jnp.bfloat16

    @pl.when(s == 0)
    def _():
        s1t3_scr[...] = _bdot(x_ref[...], w1_ref[...])

    @pl.when(s < ni)
    def _():
        i = s
        slot = jax.lax.rem(i, 2)

        @pl.when(i >= 2)
        def _():
            pltpu.make_async_copy(
                avm_scr.at[slot, pl.ds(0, bf)],
                ahbm_ref.at[pl.ds((i - 2) * bf, bf), :],
                wsem.at[slot]).wait()

        a16 = adj_ref[...].astype(bf16)
        avm_scr[slot, pl.ds(0, bf)] = a16
        pltpu.make_async_copy(
            avm_scr.at[slot, pl.ds(0, bf)],
            ahbm_ref.at[pl.ds(i * bf, bf), :],
            wsem.at[slot]).start()

        h = jnp.maximum(
            jnp.dot(a16, s1t3_scr[...].astype(bf16),
                    preferred_element_type=jnp.float32) + b1_ref[...], 0.0)
        s2_scr[pl.ds(i * bf, bf), :] = _bdot(h, w2_ref[...])
        p12_scr[pl.ds(i * bf, bf), :] = _bdot(h, lw1_ref[...])

    def _read_phase(i, use_fn):
        slot = jax.lax.rem(i, 2)

        @pl.when(i == 0)
        def _():
            pltpu.make_async_copy(
                ahbm_ref.at[pl.ds(0, bs), :],
                avm_scr.at[0], rsem.at[0]).start()

        @pl.when(i + 1 < nj)
        def _():
            nslot = jax.lax.rem(i + 1, 2)
            pltpu.make_async_copy(
                ahbm_ref.at[pl.ds((i + 1) * bs, bs), :],
                avm_scr.at[nslot], rsem.at[nslot]).start()

        pltpu.make_async_copy(
            ahbm_ref.at[pl.ds(i * bs, bs), :],
            avm_scr.at[slot], rsem.at[slot]).wait()
        use_fn(i, avm_scr[slot])

    @pl.when(jnp.logical_and(s >= ni, s < ni + nj))
    def _():
        @pl.when(s == ni)
        def _():
            for last_slot in (0, 1):
                pltpu.make_async_copy(
                    avm_scr.at[last_slot, pl.ds(0, bf)],
                    ahbm_ref.at[pl.ds(0, bf), :],
                    wsem.at[last_slot]).wait()

        def _use2(i, a16):
            x2 = jnp.maximum(
                _bdot(a16, s2_scr[...]) + b2_ref[...], 0.0)
            t3 = _bdot(_bdot(x2, w3_ref[...]), lw3_ref[...])
            s1t3_scr[pl.ds(i * bs, bs), :] = t3
            p12_scr[pl.ds(i * bs, bs), :] = (
                p12_scr[pl.ds(i * bs, bs), :] + _bdot(x2, lw2_ref[...]))

        _read_phase(s - ni, _use2)

    @pl.when(s >= ni + nj)
    def _():
        def _use3(i, a16):
            acc = _bdot(a16, s1t3_scr[...])
            cb = _bdot(b3_ref[...], lw3_ref[...])
            logits = (acc + p12_scr[pl.ds(i * bs, bs), :] + cb
                      + linb_ref[...])
            lane = jax.lax.broadcasted_iota(jnp.int32, logits.shape, 1)
            mask = lane < nclass
            neg = jnp.where(mask, logits, -1e30)
            m = jnp.max(neg, axis=1, keepdims=True)
            e = jnp.where(mask, jnp.exp(logits - m), 0.0)
            lse = jnp.log(jnp.sum(e, axis=1, keepdims=True))
            out_ref[...] = (logits - m - lse)[:, :nclass]

        _read_phase(s - ni - nj, _use3)


def kernel(x, adj, W1, b1, W2, b2, W3, b3, lin_W, lin_b):
    n, f = x.shape
    h = W1.shape[1]
    o = W3.shape[1]
    c = lin_W.shape[1]

    bf16 = jnp.bfloat16
    f32 = jnp.float32

    bf = _pick_block(n, 200)
    ni = n // bf
    bs = bf
    nj = n // bs

    pad = ((0, 0), (0, h - c))
    lw1 = jnp.pad(lin_W[:h], pad)
    lw2 = jnp.pad(lin_W[h:h + h], pad)
    lw3 = jnp.pad(lin_W[h + h:], pad)
    linb = jnp.pad(lin_b.reshape(1, c), pad)
    b1r = b1.reshape(1, h)
    b2r = b2.reshape(1, h)
    b3r = b3.reshape(1, o)

    const = lambda s: (0, 0)
    wspec = pl.BlockSpec((h, h), const)
    bspec = pl.BlockSpec((1, h), const)

    out, _ = pl.pallas_call(
        functools.partial(_mega_body, ni=ni, nj=nj, bf=bf, bs=bs, nclass=c),
        grid=(ni + 2 * nj,),
        in_specs=[pl.BlockSpec((n, f), const),
                  pl.BlockSpec((bf, n), lambda s: (jnp.minimum(s, ni - 1), 0)),
                  pl.BlockSpec((f, h), const), bspec, wspec, bspec, wspec,
                  wspec, wspec, wspec, bspec, bspec],
        out_specs=[pl.BlockSpec(
                       (bs, c),
                       lambda s: (jnp.maximum(s - ni - nj, 0), 0)),
                   pl.BlockSpec(memory_space=pltpu.MemorySpace.HBM)],
        out_shape=[jax.ShapeDtypeStruct((n, c), f32),
                   jax.ShapeDtypeStruct((n, n), bf16)],
        scratch_shapes=[pltpu.VMEM((n, h), f32), pltpu.VMEM((n, h), f32),
                        pltpu.VMEM((n, h), f32),
                        pltpu.VMEM((2, bs, n), bf16),
                        pltpu.SemaphoreType.DMA((2,)),
                        pltpu.SemaphoreType.DMA((2,))],
    )(x, adj, W1, b1r, W2, b2r, W3, lw1, lw2, lw3, b3r, linb)

    return out


# speedup vs baseline: 1.1440x; 1.1440x over previous
import functools

import jax
import jax.numpy as jnp
from jax.experimental import pallas as pl
from jax.experimental.pallas import tpu as pltpu


def _bdot(a, b):
    return jnp.dot(a.astype(jnp.bfloat16), b.astype(jnp.bfloat16),
                   preferred_element_type=jnp.float32)


def _pick_block(n, cap, step=16):
    best = 8
    for d in range(step, cap + 1, step):
        if n % d == 0:
            best = d
    return best


def _p1_body(x_ref, adj_ref, w1_ref, b1_ref, w2_ref, x1_ref, s2_ref, a16_ref,
             s1_scr):
    @pl.when(pl.program_id(0) == 0)
    def _():
        s1_scr[...] = _bdot(x_ref[...], w1_ref[...])

    a16 = adj_ref[...].astype(jnp.bfloat16)
    a16_ref[...] = a16
    h = jnp.maximum(
        jnp.dot(a16, s1_scr[...].astype(jnp.bfloat16),
                preferred_element_type=jnp.float32) + b1_ref[...], 0.0)
    x1_ref[...] = h
    s2_ref[...] = _bdot(h, w2_ref[...])


def _p23_body(a16_ref, s2_ref, x1_ref, b2_ref, w3_ref, lw1_ref, lw2_ref,
              lw3_ref, b3_ref, linb_ref, out_ref, t3_scr, p12_scr, s16_scr,
              *, bj, nclass):
    phase = pl.program_id(0)
    i = pl.program_id(1)

    @pl.when(jnp.logical_and(phase == 0, i == 0))
    def _():
        s16_scr[...] = s2_ref[...].astype(jnp.bfloat16)

    @pl.when(phase == 0)
    def _():
        x2 = jnp.maximum(
            jnp.dot(a16_ref[...], s16_scr[...],
                    preferred_element_type=jnp.float32) + b2_ref[...], 0.0)
        t3 = _bdot(_bdot(x2, w3_ref[...]), lw3_ref[...])
        t3_scr[pl.ds(i * bj, bj), :] = t3
        p12_scr[pl.ds(i * bj, bj), :] = (
            _bdot(x1_ref[...], lw1_ref[...]) + _bdot(x2, lw2_ref[...]))

    @pl.when(jnp.logical_and(phase == 1, i == 0))
    def _():
        s16_scr[...] = t3_scr[...].astype(jnp.bfloat16)

    @pl.when(phase == 1)
    def _():
        acc = jnp.dot(a16_ref[...], s16_scr[...],
                      preferred_element_type=jnp.float32)
        cb = _bdot(b3_ref[...], lw3_ref[...])
        logits = (acc + p12_scr[pl.ds(i * bj, bj), :] + cb + linb_ref[...])
        lane = jax.lax.broadcasted_iota(jnp.int32, logits.shape, 1)
        mask = lane < nclass
        neg = jnp.where(mask, logits, -1e30)
        m = jnp.max(neg, axis=1, keepdims=True)
        e = jnp.where(mask, jnp.exp(logits - m), 0.0)
        lse = jnp.log(jnp.sum(e, axis=1, keepdims=True))
        out_ref[...] = (logits - m - lse)[:, :nclass]


def kernel(x, adj, W1, b1, W2, b2, W3, b3, lin_W, lin_b):
    n, f = x.shape
    h = W1.shape[1]
    o = W3.shape[1]
    c = lin_W.shape[1]

    bi = _pick_block(n, 400)
    ni = n // bi
    bj = _pick_block(n, 1000, step=8)
    nj = n // bj

    pad = ((0, 0), (0, h - c))
    lw1 = jnp.pad(lin_W[:h], pad)
    lw2 = jnp.pad(lin_W[h:h + h], pad)
    lw3 = jnp.pad(lin_W[h + h:], pad)
    linb = jnp.pad(lin_b.reshape(1, c), pad)
    b1r = b1.reshape(1, h)
    b2r = b2.reshape(1, h)
    b3r = b3.reshape(1, o)

    f32 = jnp.float32
    adj_spec = pl.BlockSpec((bi, n), lambda i: (i, 0))
    a16_spec = pl.BlockSpec((bj, n), lambda i: (i, 0))
    row_spec = pl.BlockSpec((bi, h), lambda i: (i, 0))
    rowj_spec = pl.BlockSpec((bj, h), lambda i: (i, 0))
    full_spec = pl.BlockSpec((n, h), lambda i: (0, 0))
    w_spec = pl.BlockSpec((h, h), lambda i: (0, 0))
    bias_spec = pl.BlockSpec((1, h), lambda i: (0, 0))

    x1, s2, a16 = pl.pallas_call(
        _p1_body,
        grid=(ni,),
        in_specs=[pl.BlockSpec((n, f), lambda i: (0, 0)), adj_spec,
                  pl.BlockSpec((f, h), lambda i: (0, 0)), bias_spec, w_spec],
        out_specs=[row_spec, row_spec, adj_spec],
        out_shape=[jax.ShapeDtypeStruct((n, h), f32),
                   jax.ShapeDtypeStruct((n, h), f32),
                   jax.ShapeDtypeStruct((n, n), jnp.bfloat16)],
        scratch_shapes=[pltpu.VMEM((n, h), f32)],
    )(x, adj, W1, b1r, W2)

    out = pl.pallas_call(
        functools.partial(_p23_body, bj=bj, nclass=c),
        grid=(2, nj),
        in_specs=[pl.BlockSpec((bj, n), lambda p, i: (i, 0)),
                  pl.BlockSpec((n, h), lambda p, i: (0, 0)),
                  pl.BlockSpec((bj, h), lambda p, i: (i * (1 - p), 0)),
                  pl.BlockSpec((1, h), lambda p, i: (0, 0)),
                  pl.BlockSpec((h, h), lambda p, i: (0, 0)),
                  pl.BlockSpec((h, h), lambda p, i: (0, 0)),
                  pl.BlockSpec((h, h), lambda p, i: (0, 0)),
                  pl.BlockSpec((h, h), lambda p, i: (0, 0)),
                  pl.BlockSpec((1, h), lambda p, i: (0, 0)),
                  pl.BlockSpec((1, h), lambda p, i: (0, 0))],
        out_specs=pl.BlockSpec((bj, c), lambda p, i: (i * p, 0)),
        out_shape=jax.ShapeDtypeStruct((n, c), f32),
        scratch_shapes=[pltpu.VMEM((n, h), f32),
                        pltpu.VMEM((n, h), f32),
                        pltpu.VMEM((n, h), jnp.bfloat16)],
    )(a16, s2, x1, b2r, W3, lw1, lw2, lw3, b3r, linb)

    return out


# speedup vs baseline: 1.1538x; 1.0086x over previous
import functools

import jax
import jax.numpy as jnp
from jax.experimental import pallas as pl
from jax.experimental.pallas import tpu as pltpu


def _bdot(a, b):
    return jnp.dot(a.astype(jnp.bfloat16), b.astype(jnp.bfloat16),
                   preferred_element_type=jnp.float32)


def _pick_block(n, cap, step=16):
    best = 8
    for d in range(step, cap + 1, step):
        if n % d == 0:
            best = d
    return best


def _p1_body(x_ref, adj_ref, w1_ref, b1_ref, w2_ref, x1_ref, s2_ref, a16_ref,
             s1_scr):
    @pl.when(pl.program_id(0) == 0)
    def _():
        s1_scr[...] = _bdot(x_ref[...], w1_ref[...])

    a16 = adj_ref[...].astype(jnp.bfloat16)
    a16_ref[...] = a16
    h = jnp.maximum(
        jnp.dot(a16, s1_scr[...].astype(jnp.bfloat16),
                preferred_element_type=jnp.float32) + b1_ref[...], 0.0)
    x1_ref[...] = h.astype(jnp.bfloat16)
    s2_ref[...] = _bdot(h, w2_ref[...]).astype(jnp.bfloat16)


def _p23_body(a16_ref, s2_ref, x1_ref, b2_ref, w3_ref, lw1_ref, lw2_ref,
              lw3_ref, b3_ref, linb_ref, out_ref, t3_scr, p12_scr, s16_scr,
              *, bj, nclass):
    phase = pl.program_id(0)
    i = pl.program_id(1)

    @pl.when(phase == 0)
    def _():
        x2 = jnp.maximum(
            jnp.dot(a16_ref[...], s2_ref[...],
                    preferred_element_type=jnp.float32) + b2_ref[...], 0.0)
        t3 = _bdot(_bdot(x2, w3_ref[...]), lw3_ref[...])
        t3_scr[pl.ds(i * bj, bj), :] = t3
        p12_scr[pl.ds(i * bj, bj), :] = (
            _bdot(x1_ref[...], lw1_ref[...]) + _bdot(x2, lw2_ref[...]))

    @pl.when(jnp.logical_and(phase == 1, i == 0))
    def _():
        s16_scr[...] = t3_scr[...].astype(jnp.bfloat16)

    @pl.when(phase == 1)
    def _():
        acc = jnp.dot(a16_ref[...], s16_scr[...],
                      preferred_element_type=jnp.float32)
        cb = _bdot(b3_ref[...], lw3_ref[...])
        logits = (acc + p12_scr[pl.ds(i * bj, bj), :] + cb + linb_ref[...])
        lane = jax.lax.broadcasted_iota(jnp.int32, logits.shape, 1)
        mask = lane < nclass
        neg = jnp.where(mask, logits, -1e30)
        m = jnp.max(neg, axis=1, keepdims=True)
        e = jnp.where(mask, jnp.exp(logits - m), 0.0)
        lse = jnp.log(jnp.sum(e, axis=1, keepdims=True))
        out_ref[...] = (logits - m - lse)[:, :nclass]


def kernel(x, adj, W1, b1, W2, b2, W3, b3, lin_W, lin_b):
    n, f = x.shape
    h = W1.shape[1]
    o = W3.shape[1]
    c = lin_W.shape[1]

    bi = _pick_block(n, 400)
    ni = n // bi
    bj = _pick_block(n, 1000, step=8)
    nj = n // bj

    pad = ((0, 0), (0, h - c))
    lw1 = jnp.pad(lin_W[:h], pad)
    lw2 = jnp.pad(lin_W[h:h + h], pad)
    lw3 = jnp.pad(lin_W[h + h:], pad)
    linb = jnp.pad(lin_b.reshape(1, c), pad)
    b1r = b1.reshape(1, h)
    b2r = b2.reshape(1, h)
    b3r = b3.reshape(1, o)

    f32 = jnp.float32
    adj_spec = pl.BlockSpec((bi, n), lambda i: (i, 0))
    a16_spec = pl.BlockSpec((bj, n), lambda i: (i, 0))
    row_spec = pl.BlockSpec((bi, h), lambda i: (i, 0))
    rowj_spec = pl.BlockSpec((bj, h), lambda i: (i, 0))
    full_spec = pl.BlockSpec((n, h), lambda i: (0, 0))
    w_spec = pl.BlockSpec((h, h), lambda i: (0, 0))
    bias_spec = pl.BlockSpec((1, h), lambda i: (0, 0))

    x1, s2, a16 = pl.pallas_call(
        _p1_body,
        grid=(ni,),
        in_specs=[pl.BlockSpec((n, f), lambda i: (0, 0)), adj_spec,
                  pl.BlockSpec((f, h), lambda i: (0, 0)), bias_spec, w_spec],
        out_specs=[row_spec, row_spec, adj_spec],
        out_shape=[jax.ShapeDtypeStruct((n, h), jnp.bfloat16),
                   jax.ShapeDtypeStruct((n, h), jnp.bfloat16),
                   jax.ShapeDtypeStruct((n, n), jnp.bfloat16)],
        scratch_shapes=[pltpu.VMEM((n, h), f32)],
    )(x, adj, W1, b1r, W2)

    out = pl.pallas_call(
        functools.partial(_p23_body, bj=bj, nclass=c),
        grid=(2, nj),
        in_specs=[pl.BlockSpec((bj, n), lambda p, i: (i, 0)),
                  pl.BlockSpec((n, h), lambda p, i: (0, 0)),
                  pl.BlockSpec((bj, h), lambda p, i: (i * (1 - p), 0)),
                  pl.BlockSpec((1, h), lambda p, i: (0, 0)),
                  pl.BlockSpec((h, h), lambda p, i: (0, 0)),
                  pl.BlockSpec((h, h), lambda p, i: (0, 0)),
                  pl.BlockSpec((h, h), lambda p, i: (0, 0)),
                  pl.BlockSpec((h, h), lambda p, i: (0, 0)),
                  pl.BlockSpec((1, h), lambda p, i: (0, 0)),
                  pl.BlockSpec((1, h), lambda p, i: (0, 0))],
        out_specs=pl.BlockSpec((bj, c), lambda p, i: (i * p, 0)),
        out_shape=jax.ShapeDtypeStruct((n, c), f32),
        scratch_shapes=[pltpu.VMEM((n, h), f32),
                        pltpu.VMEM((n, h), f32),
                        pltpu.VMEM((n, h), jnp.bfloat16)],
    )(a16, s2, x1, b2r, W3, lw1, lw2, lw3, b3r, linb)

    return out
